# Initial kernel scaffold; baseline (speedup 1.0000x reference)
import functools

import jax
import jax.numpy as jnp
from jax import lax
from jax.experimental import pallas as pl
from jax.experimental.pallas import tpu as pltpu
from jax.experimental.pallas import tpu_sc as plsc

N = 10000
NPAD = 10240
D = 128
E = 320000
NC, NS, LANES = 2, 16, 16
NW = NC * NS
EPT = E // NW
CH = 80
NIT = EPT // CH
RPT = NPAD // NS
DEG_CH = 2000
DEG_NIT = EPT // DEG_CH


def _gather16(x, i):
    dnums = lax.GatherDimensionNumbers(
        offset_dims=(), collapsed_slice_dims=(0,), start_index_map=(0,))
    return lax.gather(x, i[:, None], dnums, (1,),
                      mode=lax.GatherScatterMode.PROMISE_IN_BOUNDS)


def _deg_body(src_hbm, dst_hbm, out_hbm, hist_s, hist_d, idx_v):
    c = lax.axis_index("c")
    s = lax.axis_index("s")
    wid = c * NS + s
    zeros16 = jnp.zeros((LANES,), jnp.float32)

    def zbody(i, _):
        hist_s[pl.ds(i * LANES, LANES)] = zeros16
        hist_d[pl.ds(i * LANES, LANES)] = zeros16
        return 0
    lax.fori_loop(0, NPAD // LANES, zbody, 0)

    iota = lax.iota(jnp.int32, LANES)

    def count_chunks(edge_hbm, hist):
        def chunk(k, _):
            pltpu.sync_copy(edge_hbm.at[pl.ds(wid * EPT + k * DEG_CH, DEG_CH)],
                            idx_v)

            def vec(v, _):
                idx = idx_v[pl.ds(v * LANES, LANES)]
                srt = lax.sort(idx)
                prv = _gather16(srt, jnp.maximum(iota - 1, 0))
                nxt = _gather16(srt, jnp.minimum(iota + 1, LANES - 1))
                first = (srt != prv) | (iota == 0)
                last = (srt != nxt) | (iota == LANES - 1)
                run_start = plsc.cummax(jnp.where(first, iota, 0))
                cnt = (iota - run_start + 1).astype(jnp.float32)
                plsc.addupdate_scatter(hist, [srt], cnt, mask=last)
                return 0
            lax.fori_loop(0, DEG_CH // LANES, vec, 0)
            return 0
        lax.fori_loop(0, DEG_NIT, chunk, 0)

    count_chunks(src_hbm, hist_s)
    count_chunks(dst_hbm, hist_d)
    pltpu.sync_copy(hist_s, out_hbm.at[0, wid])
    pltpu.sync_copy(hist_d, out_hbm.at[1, wid])


def _degrees(src, dst):
    mesh = plsc.VectorSubcoreMesh(core_axis_name="c", subcore_axis_name="s")
    return pl.kernel(
        _deg_body,
        out_type=jax.ShapeDtypeStruct((2, NW, NPAD), jnp.float32),
        mesh=mesh,
        scratch_types=[
            pltpu.VMEM((NPAD,), jnp.float32),
            pltpu.VMEM((NPAD,), jnp.float32),
            pltpu.VMEM((DEG_CH,), jnp.int32),
        ],
    )(src, dst)


def _agg_body(hs_hbm, src_hbm, dst_hbm, out_hbm,
              m_sh, sidx, didx, rows, zbuf, gsem):
    c = lax.axis_index("c")
    s = lax.axis_index("s")
    wid = c * NS + s
    zeros16 = jnp.zeros((LANES,), jnp.float32)

    def zb(i, _):
        zbuf[i // (D // LANES), pl.ds((i % (D // LANES)) * LANES, LANES)] = zeros16
        return 0
    lax.fori_loop(0, 128 * (D // LANES), zb, 0)
    for k in range(RPT // 128):
        pltpu.sync_copy(zbuf, m_sh.at[pl.ds(s * RPT + k * 128, 128)])
    plsc.subcore_barrier()

    base = wid * EPT

    def body(i, _):
        off = base + i * CH
        pltpu.sync_copy(src_hbm.at[pl.ds(off, CH)], sidx)
        pltpu.sync_copy(dst_hbm.at[pl.ds(off, CH)], didx)
        pltpu.async_copy(hs_hbm.at[sidx], rows, gsem).wait()
        pltpu.sync_copy(rows, m_sh.at[didx], add=True)
        return 0
    lax.fori_loop(0, NIT, body, 0)

    plsc.subcore_barrier()
    pltpu.sync_copy(m_sh.at[pl.ds(s * RPT, RPT)],
                    out_hbm.at[c, pl.ds(s * RPT, RPT)])


def _aggregate(hs, src, dst):
    mesh = plsc.VectorSubcoreMesh(core_axis_name="c", subcore_axis_name="s")
    return pl.kernel(
        _agg_body,
        out_type=jax.ShapeDtypeStruct((NC, NPAD, D), jnp.float32),
        mesh=mesh,
        scratch_types=[
            pltpu.VMEM_SHARED((NPAD, D), jnp.float32),
            pltpu.VMEM((CH,), jnp.int32),
            pltpu.VMEM((CH,), jnp.int32),
            pltpu.VMEM((CH, D), jnp.float32),
            pltpu.VMEM((128, D), jnp.float32),
            pltpu.SemaphoreType.DMA,
        ],
    )(hs, src, dst)


def _dense_body(x_ref, wu1_ref, w2_ref, wu2_ref, bu1_ref, bu2_ref, degt_ref,
                hs_ref, uo_ref):
    x = x_ref[...]
    u = jnp.maximum(jnp.dot(x, wu1_ref[...],
                            preferred_element_type=jnp.float32)
                    + bu1_ref[...], 0.0)
    od = jnp.sum(degt_ref[:, 0:NW], axis=1)
    onorm = lax.rsqrt(jnp.maximum(od, 1.0))
    hu = jnp.dot(u, w2_ref[...], preferred_element_type=jnp.float32)
    hs_ref[...] = hu * onorm[:, None]
    uo_ref[...] = jnp.maximum(jnp.dot(u, wu2_ref[...],
                                      preferred_element_type=jnp.float32)
                              + bu2_ref[...], 0.0)


def _dense(x_pad, wu1, w2, wu2, bu1, bu2, degt):
    r = 512
    grid = (NPAD // r,)
    return pl.pallas_call(
        _dense_body,
        grid=grid,
        in_specs=[
            pl.BlockSpec((r, D), lambda i: (i, 0)),
            pl.BlockSpec((D, D), lambda i: (0, 0)),
            pl.BlockSpec((D, D), lambda i: (0, 0)),
            pl.BlockSpec((D, D), lambda i: (0, 0)),
            pl.BlockSpec((1, D), lambda i: (0, 0)),
            pl.BlockSpec((1, D), lambda i: (0, 0)),
            pl.BlockSpec((r, 2 * NW), lambda i: (i, 0)),
        ],
        out_specs=[
            pl.BlockSpec((r, D), lambda i: (i, 0)),
            pl.BlockSpec((r, D), lambda i: (i, 0)),
        ],
        out_shape=[
            jax.ShapeDtypeStruct((NPAD, D), jnp.float32),
            jax.ShapeDtypeStruct((NPAD, D), jnp.float32),
        ],
    )(x_pad, wu1, w2, wu2, bu1, bu2, degt)


def _final_body(m_ref, degt_ref, b2_ref, out_ref):
    mm = m_ref[0] + m_ref[1]
    ind = jnp.sum(degt_ref[:, NW:2 * NW], axis=1)
    innorm = lax.rsqrt(jnp.maximum(ind, 1.0))
    out_ref[...] = jnp.maximum(mm * innorm[:, None] + b2_ref[...], 0.0)


def _finalize(m, degt, b2):
    r = 512
    return pl.pallas_call(
        _final_body,
        grid=(NPAD // r,),
        in_specs=[
            pl.BlockSpec((NC, r, D), lambda i: (0, i, 0)),
            pl.BlockSpec((r, 2 * NW), lambda i: (i, 0)),
            pl.BlockSpec((1, D), lambda i: (0, 0)),
        ],
        out_specs=pl.BlockSpec((r, D), lambda i: (i, 0)),
        out_shape=jax.ShapeDtypeStruct((NPAD, D), jnp.float32),
    )(m, degt, b2)


def kernel(x_user, edge_index0, edge_index1, W1, b1, W2, b2, Wu1, bu1,
           Wu2, bu2):
    src = edge_index1[0]
    dst = edge_index1[1]

    deg = _degrees(src, dst)
    degt = deg.reshape(2 * NW, NPAD).T

    x_pad = jnp.zeros((NPAD, D), jnp.float32).at[:N].set(x_user)
    hs, uo = _dense(x_pad, Wu1, W2, Wu2, bu1.reshape(1, D),
                    bu2.reshape(1, D), degt)

    m = _aggregate(hs, src, dst)
    game = _finalize(m, degt, b2.reshape(1, D))

    return (game[:N], uo[:N])


# speedup vs baseline: 5.3964x; 5.3964x over previous
import functools

import jax
import jax.numpy as jnp
from jax import lax
from jax.experimental import pallas as pl
from jax.experimental.pallas import tpu as pltpu
from jax.experimental.pallas import tpu_sc as plsc

N = 10000
NPAD = 10240
D = 128
E = 320000
NC, NS, LANES = 2, 16, 16
NW = NC * NS
EPT = E // NW
CH = 80
NIT = EPT // CH
RPT = NPAD // NS
DEG_CH = 2000
DEG_NIT = EPT // DEG_CH


def _gather16(x, i):
    dnums = lax.GatherDimensionNumbers(
        offset_dims=(), collapsed_slice_dims=(0,), start_index_map=(0,))
    return lax.gather(x, i[:, None], dnums, (1,),
                      mode=lax.GatherScatterMode.PROMISE_IN_BOUNDS)


def _deg_body(src_hbm, dst_hbm, out_hbm, hist_s, hist_d, idx_v):
    c = lax.axis_index("c")
    s = lax.axis_index("s")
    wid = c * NS + s
    zeros16 = jnp.zeros((LANES,), jnp.float32)

    def zbody(i, _):
        hist_s[pl.ds(i * LANES, LANES)] = zeros16
        hist_d[pl.ds(i * LANES, LANES)] = zeros16
        return 0
    lax.fori_loop(0, NPAD // LANES, zbody, 0)

    iota = lax.iota(jnp.int32, LANES)

    def count_chunks(edge_hbm, hist):
        def chunk(k, _):
            pltpu.sync_copy(edge_hbm.at[pl.ds(wid * EPT + k * DEG_CH, DEG_CH)],
                            idx_v)

            def vec(v, _):
                idx = idx_v[pl.ds(v * LANES, LANES)]
                srt, _ = plsc.sort_key_val(idx, idx)
                prv = _gather16(srt, jnp.maximum(iota - 1, 0))
                nxt = _gather16(srt, jnp.minimum(iota + 1, LANES - 1))
                first = (srt != prv) | (iota == 0)
                last = (srt != nxt) | (iota == LANES - 1)
                run_start = plsc.cummax(jnp.where(first, iota, 0))
                cnt = (iota - run_start + 1).astype(jnp.float32)
                plsc.addupdate_scatter(hist, [srt], cnt, mask=last)
                return 0
            lax.fori_loop(0, DEG_CH // LANES, vec, 0)
            return 0
        lax.fori_loop(0, DEG_NIT, chunk, 0)

    count_chunks(src_hbm, hist_s)
    count_chunks(dst_hbm, hist_d)
    pltpu.sync_copy(hist_s, out_hbm.at[0, wid])
    pltpu.sync_copy(hist_d, out_hbm.at[1, wid])


def _degrees(src, dst):
    mesh = plsc.VectorSubcoreMesh(core_axis_name="c", subcore_axis_name="s")
    return pl.kernel(
        _deg_body,
        out_type=jax.ShapeDtypeStruct((2, NW, NPAD), jnp.float32),
        mesh=mesh,
        compiler_params=pltpu.CompilerParams(needs_layout_passes=False),
        scratch_types=[
            pltpu.VMEM((NPAD,), jnp.float32),
            pltpu.VMEM((NPAD,), jnp.float32),
            pltpu.VMEM((DEG_CH,), jnp.int32),
        ],
    )(src, dst)


def _agg_body(hs_hbm, src_hbm, dst_hbm, out_hbm,
              m_sh, sidx, didx, rows, zbuf, gsem):
    c = lax.axis_index("c")
    s = lax.axis_index("s")
    wid = c * NS + s
    zeros16 = jnp.zeros((LANES,), jnp.float32)

    def zb(i, _):
        zbuf[i // (D // LANES), pl.ds((i % (D // LANES)) * LANES, LANES)] = zeros16
        return 0
    lax.fori_loop(0, 128 * (D // LANES), zb, 0)
    for k in range(RPT // 128):
        pltpu.sync_copy(zbuf, m_sh.at[pl.ds(s * RPT + k * 128, 128)])
    plsc.subcore_barrier()

    base = wid * EPT

    def body(i, _):
        off = base + i * CH
        pltpu.sync_copy(src_hbm.at[pl.ds(off, CH)], sidx)
        pltpu.sync_copy(dst_hbm.at[pl.ds(off, CH)], didx)
        pltpu.async_copy(hs_hbm.at[sidx], rows, gsem).wait()
        pltpu.sync_copy(rows, m_sh.at[didx], add=True)
        return 0
    lax.fori_loop(0, NIT, body, 0)

    plsc.subcore_barrier()
    pltpu.sync_copy(m_sh.at[pl.ds(s * RPT, RPT)],
                    out_hbm.at[c, pl.ds(s * RPT, RPT)])


def _aggregate(hs, src, dst):
    mesh = plsc.VectorSubcoreMesh(core_axis_name="c", subcore_axis_name="s")
    return pl.kernel(
        _agg_body,
        out_type=jax.ShapeDtypeStruct((NC, NPAD, D), jnp.float32),
        mesh=mesh,
        scratch_types=[
            pltpu.VMEM_SHARED((NPAD, D), jnp.float32),
            pltpu.VMEM((CH,), jnp.int32),
            pltpu.VMEM((CH,), jnp.int32),
            pltpu.VMEM((CH, D), jnp.float32),
            pltpu.VMEM((128, D), jnp.float32),
            pltpu.SemaphoreType.DMA,
        ],
    )(hs, src, dst)


def _dense_body(x_ref, wu1_ref, w2_ref, wu2_ref, bu1_ref, bu2_ref, degt_ref,
                hs_ref, uo_ref):
    x = x_ref[...]
    u = jnp.maximum(jnp.dot(x, wu1_ref[...],
                            preferred_element_type=jnp.float32)
                    + bu1_ref[...], 0.0)
    od = jnp.sum(degt_ref[:, 0:NW], axis=1)
    onorm = lax.rsqrt(jnp.maximum(od, 1.0))
    hu = jnp.dot(u, w2_ref[...], preferred_element_type=jnp.float32)
    hs_ref[...] = hu * onorm[:, None]
    uo_ref[...] = jnp.maximum(jnp.dot(u, wu2_ref[...],
                                      preferred_element_type=jnp.float32)
                              + bu2_ref[...], 0.0)


def _dense(x_pad, wu1, w2, wu2, bu1, bu2, degt):
    r = 512
    grid = (NPAD // r,)
    return pl.pallas_call(
        _dense_body,
        grid=grid,
        in_specs=[
            pl.BlockSpec((r, D), lambda i: (i, 0)),
            pl.BlockSpec((D, D), lambda i: (0, 0)),
            pl.BlockSpec((D, D), lambda i: (0, 0)),
            pl.BlockSpec((D, D), lambda i: (0, 0)),
            pl.BlockSpec((1, D), lambda i: (0, 0)),
            pl.BlockSpec((1, D), lambda i: (0, 0)),
            pl.BlockSpec((r, 2 * NW), lambda i: (i, 0)),
        ],
        out_specs=[
            pl.BlockSpec((r, D), lambda i: (i, 0)),
            pl.BlockSpec((r, D), lambda i: (i, 0)),
        ],
        out_shape=[
            jax.ShapeDtypeStruct((NPAD, D), jnp.float32),
            jax.ShapeDtypeStruct((NPAD, D), jnp.float32),
        ],
    )(x_pad, wu1, w2, wu2, bu1, bu2, degt)


def _final_body(m_ref, degt_ref, b2_ref, out_ref):
    mm = m_ref[0] + m_ref[1]
    ind = jnp.sum(degt_ref[:, NW:2 * NW], axis=1)
    innorm = lax.rsqrt(jnp.maximum(ind, 1.0))
    out_ref[...] = jnp.maximum(mm * innorm[:, None] + b2_ref[...], 0.0)


def _finalize(m, degt, b2):
    r = 512
    return pl.pallas_call(
        _final_body,
        grid=(NPAD // r,),
        in_specs=[
            pl.BlockSpec((NC, r, D), lambda i: (0, i, 0)),
            pl.BlockSpec((r, 2 * NW), lambda i: (i, 0)),
            pl.BlockSpec((1, D), lambda i: (0, 0)),
        ],
        out_specs=pl.BlockSpec((r, D), lambda i: (i, 0)),
        out_shape=jax.ShapeDtypeStruct((NPAD, D), jnp.float32),
    )(m, degt, b2)


def kernel(x_user, edge_index0, edge_index1, W1, b1, W2, b2, Wu1, bu1,
           Wu2, bu2):
    src = edge_index1[0]
    dst = edge_index1[1]

    deg = _degrees(src, dst)
    degt = deg.reshape(2 * NW, NPAD).T

    x_pad = jnp.zeros((NPAD, D), jnp.float32).at[:N].set(x_user)
    hs, uo = _dense(x_pad, Wu1, W2, Wu2, bu1.reshape(1, D),
                    bu2.reshape(1, D), degt)

    m = _aggregate(hs, src, dst)
    game = _finalize(m, degt, b2.reshape(1, D))

    return (game[:N], uo[:N])


# speedup vs baseline: 6.8784x; 1.2746x over previous
import jax
import jax.numpy as jnp
from jax import lax
from jax.experimental import pallas as pl
from jax.experimental.pallas import tpu as pltpu
from jax.experimental.pallas import tpu_sc as plsc

N = 10000
D = 128
E = 320000
NC, NS, LANES = 2, 16, 16
NW = NC * NS
EPT = E // NW
CH = 40
NIT = EPT // CH
MPAD = 10240
NBUF = 5
RPT = MPAD // NS
DEG_CH = 2000
DEG_NIT = EPT // DEG_CH
_SC_PARAMS = pltpu.CompilerParams(needs_layout_passes=False)


def _gather16(x, i):
    dnums = lax.GatherDimensionNumbers(
        offset_dims=(), collapsed_slice_dims=(0,), start_index_map=(0,))
    return lax.gather(x, i[:, None], dnums, (1,),
                      mode=lax.GatherScatterMode.PROMISE_IN_BOUNDS)


def _deg_body(src_hbm, dst_hbm, out_hbm, hist_s, hist_d, idx_v):
    c = lax.axis_index("c")
    s = lax.axis_index("s")
    wid = c * NS + s
    zeros16 = jnp.zeros((LANES,), jnp.float32)

    def zbody(i, _):
        hist_s[pl.ds(i * LANES, LANES)] = zeros16
        hist_d[pl.ds(i * LANES, LANES)] = zeros16
        return 0
    lax.fori_loop(0, N // LANES, zbody, 0)

    iota = lax.iota(jnp.int32, LANES)

    def count_chunks(edge_hbm, hist):
        def chunk(k, _):
            pltpu.sync_copy(edge_hbm.at[pl.ds(wid * EPT + k * DEG_CH, DEG_CH)],
                            idx_v)

            def vec(v, _):
                idx = idx_v[pl.ds(v * LANES, LANES)]
                srt, _unused = plsc.sort_key_val(idx, idx)
                prv = _gather16(srt, jnp.maximum(iota - 1, 0))
                nxt = _gather16(srt, jnp.minimum(iota + 1, LANES - 1))
                first = (srt != prv) | (iota == 0)
                last = (srt != nxt) | (iota == LANES - 1)
                run_start = plsc.cummax(jnp.where(first, iota, 0))
                cnt = (iota - run_start + 1).astype(jnp.float32)
                plsc.addupdate_scatter(hist, [srt], cnt, mask=last)
                return 0
            lax.fori_loop(0, DEG_CH // LANES, vec, 0)
            return 0
        lax.fori_loop(0, DEG_NIT, chunk, 0)

    count_chunks(src_hbm, hist_s)
    count_chunks(dst_hbm, hist_d)
    pltpu.sync_copy(hist_s, out_hbm.at[0, wid])
    pltpu.sync_copy(hist_d, out_hbm.at[1, wid])


def _degrees(src, dst):
    mesh = plsc.VectorSubcoreMesh(core_axis_name="c", subcore_axis_name="s")
    return pl.kernel(
        _deg_body,
        out_type=jax.ShapeDtypeStruct((2, NW, N), jnp.float32),
        mesh=mesh,
        compiler_params=_SC_PARAMS,
        scratch_types=[
            pltpu.VMEM((N,), jnp.float32),
            pltpu.VMEM((N,), jnp.float32),
            pltpu.VMEM((DEG_CH,), jnp.int32),
        ],
    )(src, dst)


def _agg_body(hs_hbm, ei_hbm, out_hbm,
              m_sh, sidxb, didxb, rows, zbuf, isem, gsem, ssem):
    c = lax.axis_index("c")
    s = lax.axis_index("s")
    wid = c * NS + s
    zeros16 = jnp.zeros((LANES,), jnp.float32)

    def zb(i, _):
        zbuf[i // (D // LANES), pl.ds((i % (D // LANES)) * LANES, LANES)] = zeros16
        return 0
    lax.fori_loop(0, 64 * (D // LANES), zb, 0)
    for k in range(RPT // 64):
        pltpu.sync_copy(zbuf, m_sh.at[pl.ds(s * RPT + k * 64, 64)])
    plsc.subcore_barrier()

    def _load(i, b):
        pltpu.async_copy(ei_hbm.at[wid, i, 0], sidxb.at[b], isem.at[b])
        pltpu.async_copy(ei_hbm.at[wid, i, 1], didxb.at[b], isem.at[b])

    def _gather(b):
        pltpu.async_copy(hs_hbm.at[sidxb.at[b]], rows.at[b], gsem.at[b])

    def _scatter(b):
        pltpu.async_copy(rows.at[b], m_sh.at[didxb.at[b]], ssem.at[b],
                         add=True)

    def _drain_rows(sem, b):
        pltpu.make_async_copy(hs_hbm.at[pl.ds(0, CH)], rows.at[b],
                              sem.at[b]).wait()

    def _drain_idx(b):
        pltpu.make_async_copy(ei_hbm.at[wid, 0, 0], sidxb.at[b],
                              isem.at[b]).wait()
        pltpu.make_async_copy(ei_hbm.at[wid, 0, 1], didxb.at[b],
                              isem.at[b]).wait()

    _load(0, 0)
    _load(1, 1)
    _drain_idx(0)
    _gather(0)

    def rnd(r, _):
        for b in range(NBUF):
            i = r * NBUF + b
            b1 = (b + 1) % NBUF
            b2 = (b + 2) % NBUF
            _drain_rows(gsem, b)
            _scatter(b)
            def prefetch():
                pl.when(i >= NBUF - 2)(lambda: _drain_rows(ssem, b2))
                _load(i + 2, b2)
            pl.when(i + 2 < NIT)(prefetch)
            def launch_gather():
                _drain_idx(b1)
                _gather(b1)
            pl.when(i + 1 < NIT)(launch_gather)
        return 0
    lax.fori_loop(0, NIT // NBUF, rnd, 0)

    for b in range(NBUF):
        _drain_rows(ssem, b)

    plsc.subcore_barrier()
    pltpu.sync_copy(m_sh.at[pl.ds(s * RPT, RPT)],
                    out_hbm.at[c, pl.ds(s * RPT, RPT)])


def _aggregate(hs, ei):
    mesh = plsc.VectorSubcoreMesh(core_axis_name="c", subcore_axis_name="s")
    return pl.kernel(
        _agg_body,
        out_type=jax.ShapeDtypeStruct((NC, MPAD, D), jnp.float32),
        mesh=mesh,
        compiler_params=_SC_PARAMS,
        scratch_types=[
            pltpu.VMEM_SHARED((MPAD, D), jnp.float32),
            pltpu.VMEM((NBUF, CH), jnp.int32),
            pltpu.VMEM((NBUF, CH), jnp.int32),
            pltpu.VMEM((NBUF, CH, D), jnp.float32),
            pltpu.VMEM((64, D), jnp.float32),
            pltpu.SemaphoreType.DMA((NBUF,)),
            pltpu.SemaphoreType.DMA((NBUF,)),
            pltpu.SemaphoreType.DMA((NBUF,)),
        ],
    )(hs, ei)


def _dense_body(x_ref, wu1_ref, w2_ref, wu2_ref, bu1_ref, bu2_ref, degt_ref,
                hs_ref, uo_ref):
    x = x_ref[...]
    u = jnp.maximum(jnp.dot(x, wu1_ref[...],
                            preferred_element_type=jnp.float32)
                    + bu1_ref[...], 0.0)
    od = jnp.sum(degt_ref[:, 0:NW], axis=1)
    onorm = lax.rsqrt(jnp.maximum(od, 1.0))
    hu = jnp.dot(u, w2_ref[...], preferred_element_type=jnp.float32)
    hs_ref[...] = hu * onorm[:, None]
    uo_ref[...] = jnp.maximum(jnp.dot(u, wu2_ref[...],
                                      preferred_element_type=jnp.float32)
                              + bu2_ref[...], 0.0)


def _dense(x, wu1, w2, wu2, bu1, bu2, degt):
    r = 1000
    return pl.pallas_call(
        _dense_body,
        grid=(N // r,),
        in_specs=[
            pl.BlockSpec((r, D), lambda i: (i, 0)),
            pl.BlockSpec((D, D), lambda i: (0, 0)),
            pl.BlockSpec((D, D), lambda i: (0, 0)),
            pl.BlockSpec((D, D), lambda i: (0, 0)),
            pl.BlockSpec((1, D), lambda i: (0, 0)),
            pl.BlockSpec((1, D), lambda i: (0, 0)),
            pl.BlockSpec((r, 2 * NW), lambda i: (i, 0)),
        ],
        out_specs=[
            pl.BlockSpec((r, D), lambda i: (i, 0)),
            pl.BlockSpec((r, D), lambda i: (i, 0)),
        ],
        out_shape=[
            jax.ShapeDtypeStruct((N, D), jnp.float32),
            jax.ShapeDtypeStruct((N, D), jnp.float32),
        ],
    )(x, wu1, w2, wu2, bu1, bu2, degt)


def _final_body(m_ref, degt_ref, b2_ref, out_ref):
    mm = m_ref[0] + m_ref[1]
    ind = jnp.sum(degt_ref[:, NW:2 * NW], axis=1)
    innorm = lax.rsqrt(jnp.maximum(ind, 1.0))
    out_ref[...] = jnp.maximum(mm * innorm[:, None] + b2_ref[...], 0.0)


def _finalize(m, degt, b2):
    r = 1000
    return pl.pallas_call(
        _final_body,
        grid=(N // r,),
        in_specs=[
            pl.BlockSpec((NC, r, D), lambda i: (0, i, 0)),
            pl.BlockSpec((r, 2 * NW), lambda i: (i, 0)),
            pl.BlockSpec((1, D), lambda i: (0, 0)),
        ],
        out_specs=pl.BlockSpec((r, D), lambda i: (i, 0)),
        out_shape=jax.ShapeDtypeStruct((N, D), jnp.float32),
    )(m, degt, b2)


def kernel(x_user, edge_index0, edge_index1, W1, b1, W2, b2, Wu1, bu1,
           Wu2, bu2):
    src = edge_index1[0]
    dst = edge_index1[1]

    deg = _degrees(src, dst)
    degt = deg.reshape(2 * NW, N).T

    hs, uo = _dense(x_user, Wu1, W2, Wu2, bu1.reshape(1, D),
                    bu2.reshape(1, D), degt)

    ei = jnp.stack([src.reshape(NW, NIT, CH),
                    dst.reshape(NW, NIT, CH)], axis=2)
    m = _aggregate(hs, ei)
    game = _finalize(m, degt, b2.reshape(1, D))

    return (game, uo)


# speedup vs baseline: 9.4906x; 1.3798x over previous
import jax
import jax.numpy as jnp
from jax import lax
from jax.experimental import pallas as pl
from jax.experimental.pallas import tpu as pltpu
from jax.experimental.pallas import tpu_sc as plsc

N = 10000
D = 128
E = 320000
NC, NS, LANES = 2, 16, 16
NW = NC * NS
EPT = E // NW
CH = 40
NIT = EPT // CH
MPAD = 10240
NBUF = 5
RPT = MPAD // NS
DEG_CH = 2000
DEG_NIT = EPT // DEG_CH
_SC_PARAMS = pltpu.CompilerParams(needs_layout_passes=False)


def _gather16(x, i):
    dnums = lax.GatherDimensionNumbers(
        offset_dims=(), collapsed_slice_dims=(0,), start_index_map=(0,))
    return lax.gather(x, i[:, None], dnums, (1,),
                      mode=lax.GatherScatterMode.PROMISE_IN_BOUNDS)


def _deg_body(src_hbm, dst_hbm, out_hbm, hist_s, hist_d, idx_v):
    c = lax.axis_index("c")
    s = lax.axis_index("s")
    wid = c * NS + s
    zeros16 = jnp.zeros((LANES,), jnp.float32)

    def zbody(i, _):
        hist_s[pl.ds(i * LANES, LANES)] = zeros16
        hist_d[pl.ds(i * LANES, LANES)] = zeros16
        return 0
    lax.fori_loop(0, N // LANES, zbody, 0)

    iota = lax.iota(jnp.int32, LANES)

    def count_chunks(edge_hbm, hist):
        def chunk(k, _):
            pltpu.sync_copy(edge_hbm.at[pl.ds(wid * EPT + k * DEG_CH, DEG_CH)],
                            idx_v)

            def vec(v, _):
                idx = idx_v[pl.ds(v * LANES, LANES)]
                srt, _unused = plsc.sort_key_val(idx, idx)
                prv = _gather16(srt, jnp.maximum(iota - 1, 0))
                nxt = _gather16(srt, jnp.minimum(iota + 1, LANES - 1))
                first = (srt != prv) | (iota == 0)
                last = (srt != nxt) | (iota == LANES - 1)
                run_start = plsc.cummax(jnp.where(first, iota, 0))
                cnt = (iota - run_start + 1).astype(jnp.float32)
                plsc.addupdate_scatter(hist, [srt], cnt, mask=last)
                return 0
            lax.fori_loop(0, DEG_CH // LANES, vec, 0)
            return 0
        lax.fori_loop(0, DEG_NIT, chunk, 0)

    count_chunks(src_hbm, hist_s)
    count_chunks(dst_hbm, hist_d)
    pltpu.sync_copy(hist_s, out_hbm.at[0, wid])
    pltpu.sync_copy(hist_d, out_hbm.at[1, wid])


def _degrees(src, dst):
    mesh = plsc.VectorSubcoreMesh(core_axis_name="c", subcore_axis_name="s")
    return pl.kernel(
        _deg_body,
        out_type=jax.ShapeDtypeStruct((2, NW, N), jnp.float32),
        mesh=mesh,
        compiler_params=_SC_PARAMS,
        scratch_types=[
            pltpu.VMEM((N,), jnp.float32),
            pltpu.VMEM((N,), jnp.float32),
            pltpu.VMEM((DEG_CH,), jnp.int32),
        ],
    )(src, dst)


def _agg_body(hs_hbm, ei_hbm, out_hbm,
              m_sh, sidxb, didxb, rows, zbuf, isem, gsem, ssem):
    c = lax.axis_index("c")
    s = lax.axis_index("s")
    wid = c * NS + s
    zeros16 = jnp.zeros((LANES,), jnp.float32)

    def zb(i, _):
        zbuf[i // (D // LANES), pl.ds((i % (D // LANES)) * LANES, LANES)] = zeros16
        return 0
    lax.fori_loop(0, 64 * (D // LANES), zb, 0)
    for k in range(RPT // 64):
        pltpu.sync_copy(zbuf, m_sh.at[pl.ds(s * RPT + k * 64, 64)])
    plsc.subcore_barrier()

    def _load(i, b):
        pltpu.async_copy(ei_hbm.at[wid, i, 0], sidxb.at[b], isem.at[b])
        pltpu.async_copy(ei_hbm.at[wid, i, 1], didxb.at[b], isem.at[b])

    def _gather(b):
        pltpu.async_copy(hs_hbm.at[sidxb.at[b]], rows.at[b], gsem.at[b])

    def _scatter(b):
        pltpu.async_copy(rows.at[b], m_sh.at[didxb.at[b]], ssem.at[b],
                         add=True)

    def _drain_rows(sem, b):
        pltpu.make_async_copy(hs_hbm.at[pl.ds(0, CH)], rows.at[b],
                              sem.at[b]).wait()

    def _drain_idx(b):
        pltpu.make_async_copy(ei_hbm.at[wid, 0, 0], sidxb.at[b],
                              isem.at[b]).wait()
        pltpu.make_async_copy(ei_hbm.at[wid, 0, 1], didxb.at[b],
                              isem.at[b]).wait()

    _load(0, 0)
    _load(1, 1)
    _load(2, 2)
    _drain_idx(0)
    _gather(0)
    _drain_idx(1)
    _gather(1)

    def rnd(r, _):
        for b in range(NBUF):
            i = r * NBUF + b
            b2 = (b + 2) % NBUF
            b3 = (b + 3) % NBUF
            _drain_rows(gsem, b)
            _scatter(b)
            def prefetch():
                pl.when(i >= 2)(lambda: _drain_rows(ssem, b3))
                _load(i + 3, b3)
            pl.when(i + 3 < NIT)(prefetch)
            def launch_gather():
                _drain_idx(b2)
                _gather(b2)
            pl.when(i + 2 < NIT)(launch_gather)
        return 0
    lax.fori_loop(0, NIT // NBUF, rnd, 0)

    for b in range(NBUF):
        _drain_rows(ssem, b)

    plsc.subcore_barrier()
    pltpu.sync_copy(m_sh.at[pl.ds(s * RPT, RPT)],
                    out_hbm.at[c, pl.ds(s * RPT, RPT)])


def _aggregate(hs, ei):
    mesh = plsc.VectorSubcoreMesh(core_axis_name="c", subcore_axis_name="s")
    return pl.kernel(
        _agg_body,
        out_type=jax.ShapeDtypeStruct((NC, MPAD, D), jnp.float32),
        mesh=mesh,
        compiler_params=_SC_PARAMS,
        scratch_types=[
            pltpu.VMEM_SHARED((MPAD, D), jnp.float32),
            pltpu.VMEM((NBUF, CH), jnp.int32),
            pltpu.VMEM((NBUF, CH), jnp.int32),
            pltpu.VMEM((NBUF, CH, D), jnp.float32),
            pltpu.VMEM((64, D), jnp.float32),
            pltpu.SemaphoreType.DMA((NBUF,)),
            pltpu.SemaphoreType.DMA((NBUF,)),
            pltpu.SemaphoreType.DMA((NBUF,)),
        ],
    )(hs, ei)


def _dense_body(x_ref, wu1_ref, w2_ref, wu2_ref, bu1_ref, bu2_ref,
                hu_ref, uo_ref):
    x = x_ref[...]
    u = jnp.maximum(jnp.dot(x, wu1_ref[...],
                            preferred_element_type=jnp.float32)
                    + bu1_ref[...], 0.0)
    hu_ref[...] = jnp.dot(u, w2_ref[...], preferred_element_type=jnp.float32)
    uo_ref[...] = jnp.maximum(jnp.dot(u, wu2_ref[...],
                                      preferred_element_type=jnp.float32)
                              + bu2_ref[...], 0.0)


def _dense(x, wu1, w2, wu2, bu1, bu2):
    r = 1000
    return pl.pallas_call(
        _dense_body,
        grid=(N // r,),
        in_specs=[
            pl.BlockSpec((r, D), lambda i: (i, 0)),
            pl.BlockSpec((D, D), lambda i: (0, 0)),
            pl.BlockSpec((D, D), lambda i: (0, 0)),
            pl.BlockSpec((D, D), lambda i: (0, 0)),
            pl.BlockSpec((1, D), lambda i: (0, 0)),
            pl.BlockSpec((1, D), lambda i: (0, 0)),
        ],
        out_specs=[
            pl.BlockSpec((r, D), lambda i: (i, 0)),
            pl.BlockSpec((r, D), lambda i: (i, 0)),
        ],
        out_shape=[
            jax.ShapeDtypeStruct((N, D), jnp.float32),
            jax.ShapeDtypeStruct((N, D), jnp.float32),
        ],
    )(x, wu1, w2, wu2, bu1, bu2)


def _scale_body(hu_ref, degt_ref, hs_ref):
    od = jnp.sum(degt_ref[:, 0:NW], axis=1)
    onorm = lax.rsqrt(jnp.maximum(od, 1.0))
    hs_ref[...] = hu_ref[...] * onorm[:, None]


def _scale(hu, degt):
    r = 1000
    return pl.pallas_call(
        _scale_body,
        grid=(N // r,),
        in_specs=[
            pl.BlockSpec((r, D), lambda i: (i, 0)),
            pl.BlockSpec((r, 2 * NW), lambda i: (i, 0)),
        ],
        out_specs=pl.BlockSpec((r, D), lambda i: (i, 0)),
        out_shape=jax.ShapeDtypeStruct((N, D), jnp.float32),
    )(hu, degt)


def _final_body(m_ref, degt_ref, b2_ref, out_ref):
    mm = m_ref[0] + m_ref[1]
    ind = jnp.sum(degt_ref[:, NW:2 * NW], axis=1)
    innorm = lax.rsqrt(jnp.maximum(ind, 1.0))
    out_ref[...] = jnp.maximum(mm * innorm[:, None] + b2_ref[...], 0.0)


def _finalize(m, degt, b2):
    r = 1000
    return pl.pallas_call(
        _final_body,
        grid=(N // r,),
        in_specs=[
            pl.BlockSpec((NC, r, D), lambda i: (0, i, 0)),
            pl.BlockSpec((r, 2 * NW), lambda i: (i, 0)),
            pl.BlockSpec((1, D), lambda i: (0, 0)),
        ],
        out_specs=pl.BlockSpec((r, D), lambda i: (i, 0)),
        out_shape=jax.ShapeDtypeStruct((N, D), jnp.float32),
    )(m, degt, b2)


def kernel(x_user, edge_index0, edge_index1, W1, b1, W2, b2, Wu1, bu1,
           Wu2, bu2):
    src = edge_index1[0]
    dst = edge_index1[1]

    deg = _degrees(src, dst)
    degt = deg.reshape(2 * NW, N).T

    hu, uo = _dense(x_user, Wu1, W2, Wu2, bu1.reshape(1, D),
                    bu2.reshape(1, D))
    hs = _scale(hu, degt)

    ei = jnp.stack([src.reshape(NW, NIT, CH),
                    dst.reshape(NW, NIT, CH)], axis=2)
    m = _aggregate(hs, ei)
    game = _finalize(m, degt, b2.reshape(1, D))

    return (game, uo)


# speedup vs baseline: 9.4955x; 1.0005x over previous
import jax
import jax.numpy as jnp
from jax import lax
from jax.experimental import pallas as pl
from jax.experimental.pallas import tpu as pltpu
from jax.experimental.pallas import tpu_sc as plsc

N = 10000
D = 128
E = 320000
NC, NS, LANES = 2, 16, 16
NW = NC * NS
EPT = E // NW
CH = 40
NIT = EPT // CH
MPAD = 10240
NBUF = 5
RPT = MPAD // NS
DEG_CH = 2000
DEG_NIT = EPT // DEG_CH
_SC_PARAMS = pltpu.CompilerParams(needs_layout_passes=False)


def _gather16(x, i):
    dnums = lax.GatherDimensionNumbers(
        offset_dims=(), collapsed_slice_dims=(0,), start_index_map=(0,))
    return lax.gather(x, i[:, None], dnums, (1,),
                      mode=lax.GatherScatterMode.PROMISE_IN_BOUNDS)


def _deg_body(src_hbm, dst_hbm, out_hbm, hist_s, hist_d, idx_v):
    c = lax.axis_index("c")
    s = lax.axis_index("s")
    wid = c * NS + s
    zeros16 = jnp.zeros((LANES,), jnp.float32)

    def zbody(i, _):
        hist_s[pl.ds(i * LANES, LANES)] = zeros16
        hist_d[pl.ds(i * LANES, LANES)] = zeros16
        return 0
    lax.fori_loop(0, N // LANES, zbody, 0)

    ones16 = jnp.ones((LANES,), jnp.float32)

    def count_chunks(edge_hbm, hist):
        def chunk(k, _):
            pltpu.sync_copy(edge_hbm.at[pl.ds(wid * EPT + k * DEG_CH, DEG_CH)],
                            idx_v)

            def vec(v, _):
                idx = idx_v[pl.ds(v * LANES, LANES)]
                plsc.addupdate_scatter(hist, [idx], ones16)
                return 0
            lax.fori_loop(0, DEG_CH // LANES, vec, 0)
            return 0
        lax.fori_loop(0, DEG_NIT, chunk, 0)

    count_chunks(src_hbm, hist_s)
    count_chunks(dst_hbm, hist_d)
    pltpu.sync_copy(hist_s, out_hbm.at[0, wid])
    pltpu.sync_copy(hist_d, out_hbm.at[1, wid])


def _degrees(src, dst):
    mesh = plsc.VectorSubcoreMesh(core_axis_name="c", subcore_axis_name="s")
    return pl.kernel(
        _deg_body,
        out_type=jax.ShapeDtypeStruct((2, NW, N), jnp.float32),
        mesh=mesh,
        compiler_params=_SC_PARAMS,
        scratch_types=[
            pltpu.VMEM((N,), jnp.float32),
            pltpu.VMEM((N,), jnp.float32),
            pltpu.VMEM((DEG_CH,), jnp.int32),
        ],
    )(src, dst)


def _agg_body(hs_hbm, ei_hbm, out_hbm,
              m_sh, sidxb, didxb, rows, zbuf, isem, gsem, ssem):
    c = lax.axis_index("c")
    s = lax.axis_index("s")
    wid = c * NS + s
    zeros16 = jnp.zeros((LANES,), jnp.float32)

    def zb(i, _):
        zbuf[i // (D // LANES), pl.ds((i % (D // LANES)) * LANES, LANES)] = zeros16
        return 0
    lax.fori_loop(0, 64 * (D // LANES), zb, 0)
    for k in range(RPT // 64):
        pltpu.sync_copy(zbuf, m_sh.at[pl.ds(s * RPT + k * 64, 64)])
    plsc.subcore_barrier()

    def _load(i, b):
        pltpu.async_copy(ei_hbm.at[wid, i, 0], sidxb.at[b], isem.at[b])
        pltpu.async_copy(ei_hbm.at[wid, i, 1], didxb.at[b], isem.at[b])

    def _gather(b):
        pltpu.async_copy(hs_hbm.at[sidxb.at[b]], rows.at[b], gsem.at[b])

    def _scatter(b):
        pltpu.async_copy(rows.at[b], m_sh.at[didxb.at[b]], ssem.at[b],
                         add=True)

    def _drain_rows(sem, b):
        pltpu.make_async_copy(hs_hbm.at[pl.ds(0, CH)], rows.at[b],
                              sem.at[b]).wait()

    def _drain_idx(b):
        pltpu.make_async_copy(ei_hbm.at[wid, 0, 0], sidxb.at[b],
                              isem.at[b]).wait()
        pltpu.make_async_copy(ei_hbm.at[wid, 0, 1], didxb.at[b],
                              isem.at[b]).wait()

    _load(0, 0)
    _load(1, 1)
    _load(2, 2)
    _drain_idx(0)
    _gather(0)
    _drain_idx(1)
    _gather(1)

    def rnd(r, _):
        for b in range(NBUF):
            i = r * NBUF + b
            b2 = (b + 2) % NBUF
            b3 = (b + 3) % NBUF
            _drain_rows(gsem, b)
            _scatter(b)
            def prefetch():
                pl.when(i >= 2)(lambda: _drain_rows(ssem, b3))
                _load(i + 3, b3)
            pl.when(i + 3 < NIT)(prefetch)
            def launch_gather():
                _drain_idx(b2)
                _gather(b2)
            pl.when(i + 2 < NIT)(launch_gather)
        return 0
    lax.fori_loop(0, NIT // NBUF, rnd, 0)

    for b in range(NBUF):
        _drain_rows(ssem, b)

    plsc.subcore_barrier()
    pltpu.sync_copy(m_sh.at[pl.ds(s * RPT, RPT)],
                    out_hbm.at[c, pl.ds(s * RPT, RPT)])


def _aggregate(hs, ei):
    mesh = plsc.VectorSubcoreMesh(core_axis_name="c", subcore_axis_name="s")
    return pl.kernel(
        _agg_body,
        out_type=jax.ShapeDtypeStruct((NC, MPAD, D), jnp.float32),
        mesh=mesh,
        compiler_params=_SC_PARAMS,
        scratch_types=[
            pltpu.VMEM_SHARED((MPAD, D), jnp.float32),
            pltpu.VMEM((NBUF, CH), jnp.int32),
            pltpu.VMEM((NBUF, CH), jnp.int32),
            pltpu.VMEM((NBUF, CH, D), jnp.float32),
            pltpu.VMEM((64, D), jnp.float32),
            pltpu.SemaphoreType.DMA((NBUF,)),
            pltpu.SemaphoreType.DMA((NBUF,)),
            pltpu.SemaphoreType.DMA((NBUF,)),
        ],
    )(hs, ei)


def _dense_body(x_ref, wu1_ref, w2_ref, wu2_ref, bu1_ref, bu2_ref,
                hu_ref, uo_ref):
    x = x_ref[...]
    u = jnp.maximum(jnp.dot(x, wu1_ref[...],
                            preferred_element_type=jnp.float32)
                    + bu1_ref[...], 0.0)
    hu_ref[...] = jnp.dot(u, w2_ref[...], preferred_element_type=jnp.float32)
    uo_ref[...] = jnp.maximum(jnp.dot(u, wu2_ref[...],
                                      preferred_element_type=jnp.float32)
                              + bu2_ref[...], 0.0)


def _dense(x, wu1, w2, wu2, bu1, bu2):
    r = 1000
    return pl.pallas_call(
        _dense_body,
        grid=(N // r,),
        in_specs=[
            pl.BlockSpec((r, D), lambda i: (i, 0)),
            pl.BlockSpec((D, D), lambda i: (0, 0)),
            pl.BlockSpec((D, D), lambda i: (0, 0)),
            pl.BlockSpec((D, D), lambda i: (0, 0)),
            pl.BlockSpec((1, D), lambda i: (0, 0)),
            pl.BlockSpec((1, D), lambda i: (0, 0)),
        ],
        out_specs=[
            pl.BlockSpec((r, D), lambda i: (i, 0)),
            pl.BlockSpec((r, D), lambda i: (i, 0)),
        ],
        out_shape=[
            jax.ShapeDtypeStruct((N, D), jnp.float32),
            jax.ShapeDtypeStruct((N, D), jnp.float32),
        ],
    )(x, wu1, w2, wu2, bu1, bu2)


def _scale_body(hu_ref, degt_ref, hs_ref):
    od = jnp.sum(degt_ref[:, 0:NW], axis=1)
    onorm = lax.rsqrt(jnp.maximum(od, 1.0))
    hs_ref[...] = hu_ref[...] * onorm[:, None]


def _scale(hu, degt):
    r = 1000
    return pl.pallas_call(
        _scale_body,
        grid=(N // r,),
        in_specs=[
            pl.BlockSpec((r, D), lambda i: (i, 0)),
            pl.BlockSpec((r, 2 * NW), lambda i: (i, 0)),
        ],
        out_specs=pl.BlockSpec((r, D), lambda i: (i, 0)),
        out_shape=jax.ShapeDtypeStruct((N, D), jnp.float32),
    )(hu, degt)


def _final_body(m_ref, degt_ref, b2_ref, out_ref):
    mm = m_ref[0] + m_ref[1]
    ind = jnp.sum(degt_ref[:, NW:2 * NW], axis=1)
    innorm = lax.rsqrt(jnp.maximum(ind, 1.0))
    out_ref[...] = jnp.maximum(mm * innorm[:, None] + b2_ref[...], 0.0)


def _finalize(m, degt, b2):
    r = 1000
    return pl.pallas_call(
        _final_body,
        grid=(N // r,),
        in_specs=[
            pl.BlockSpec((NC, r, D), lambda i: (0, i, 0)),
            pl.BlockSpec((r, 2 * NW), lambda i: (i, 0)),
            pl.BlockSpec((1, D), lambda i: (0, 0)),
        ],
        out_specs=pl.BlockSpec((r, D), lambda i: (i, 0)),
        out_shape=jax.ShapeDtypeStruct((N, D), jnp.float32),
    )(m, degt, b2)


def kernel(x_user, edge_index0, edge_index1, W1, b1, W2, b2, Wu1, bu1,
           Wu2, bu2):
    src = edge_index1[0]
    dst = edge_index1[1]

    deg = _degrees(src, dst)
    degt = deg.reshape(2 * NW, N).T

    hu, uo = _dense(x_user, Wu1, W2, Wu2, bu1.reshape(1, D),
                    bu2.reshape(1, D))
    hs = _scale(hu, degt)

    ei = jnp.stack([src.reshape(NW, NIT, CH),
                    dst.reshape(NW, NIT, CH)], axis=2)
    m = _aggregate(hs, ei)
    game = _finalize(m, degt, b2.reshape(1, D))

    return (game, uo)


# speedup vs baseline: 11.7141x; 1.2336x over previous
import jax
import jax.numpy as jnp
from jax import lax
from jax.experimental import pallas as pl
from jax.experimental.pallas import tpu as pltpu
from jax.experimental.pallas import tpu_sc as plsc

N = 10000
D = 128
E = 320000
NC, NS, LANES = 2, 16, 16
NW = NC * NS
EPT = E // NW
CH = 40
NIT = EPT // CH
MPAD = 10240
NBUF = 5
RPT = MPAD // NS
DEG_CH = 2000
DEG_NIT = EPT // DEG_CH
_SC_PARAMS = pltpu.CompilerParams(needs_layout_passes=False)


def _gather16(x, i):
    dnums = lax.GatherDimensionNumbers(
        offset_dims=(), collapsed_slice_dims=(0,), start_index_map=(0,))
    return lax.gather(x, i[:, None], dnums, (1,),
                      mode=lax.GatherScatterMode.PROMISE_IN_BOUNDS)


def _deg_body(src_hbm, dst_hbm, out_hbm, hist_s, hist_d, idx_v):
    c = lax.axis_index("c")
    s = lax.axis_index("s")
    wid = c * NS + s
    zeros16 = jnp.zeros((LANES,), jnp.float32)

    def zbody(i, _):
        hist_s[pl.ds(i * LANES, LANES)] = zeros16
        hist_d[pl.ds(i * LANES, LANES)] = zeros16
        return 0
    lax.fori_loop(0, N // LANES, zbody, 0)

    ones16 = jnp.ones((LANES,), jnp.float32)

    def count_chunks(edge_hbm, hist):
        def chunk(k, _):
            pltpu.sync_copy(edge_hbm.at[pl.ds(wid * EPT + k * DEG_CH, DEG_CH)],
                            idx_v)

            def vec(v, _):
                idx = idx_v[pl.ds(v * LANES, LANES)]
                plsc.addupdate_scatter(hist, [idx], ones16)
                return 0
            lax.fori_loop(0, DEG_CH // LANES, vec, 0)
            return 0
        lax.fori_loop(0, DEG_NIT, chunk, 0)

    count_chunks(src_hbm, hist_s)
    count_chunks(dst_hbm, hist_d)
    pltpu.sync_copy(hist_s, out_hbm.at[0, wid])
    pltpu.sync_copy(hist_d, out_hbm.at[1, wid])


def _degrees(src, dst):
    mesh = plsc.VectorSubcoreMesh(core_axis_name="c", subcore_axis_name="s")
    return pl.kernel(
        _deg_body,
        out_type=jax.ShapeDtypeStruct((2, NW, N), jnp.float32),
        mesh=mesh,
        compiler_params=_SC_PARAMS,
        scratch_types=[
            pltpu.VMEM((N,), jnp.float32),
            pltpu.VMEM((N,), jnp.float32),
            pltpu.VMEM((DEG_CH,), jnp.int32),
        ],
    )(src, dst)


def _agg_body(hs_hbm, src_hbm, dst_hbm, out_hbm,
              m_sh, sidxb, didxb, rows, zbuf, isem, gsem, ssem):
    c = lax.axis_index("c")
    s = lax.axis_index("s")
    wid = c * NS + s
    zeros16 = jnp.zeros((LANES,), jnp.float32)

    def zb(i, _):
        zbuf[i // (D // LANES), pl.ds((i % (D // LANES)) * LANES, LANES)] = zeros16
        return 0
    lax.fori_loop(0, 64 * (D // LANES), zb, 0)
    for k in range(RPT // 64):
        pltpu.sync_copy(zbuf, m_sh.at[pl.ds(s * RPT + k * 64, 64)])
    plsc.subcore_barrier()

    def _load(i, b):
        pltpu.async_copy(src_hbm.at[wid, i], sidxb.at[b], isem.at[b])
        pltpu.async_copy(dst_hbm.at[wid, i], didxb.at[b], isem.at[b])

    def _gather(b):
        pltpu.async_copy(hs_hbm.at[sidxb.at[b]], rows.at[b], gsem.at[b])

    def _scatter(b):
        pltpu.async_copy(rows.at[b], m_sh.at[didxb.at[b]], ssem.at[b],
                         add=True)

    def _drain_rows(sem, b):
        pltpu.make_async_copy(hs_hbm.at[pl.ds(0, CH)], rows.at[b],
                              sem.at[b]).wait()

    def _drain_idx(b):
        pltpu.make_async_copy(src_hbm.at[wid, 0], sidxb.at[b],
                              isem.at[b]).wait()
        pltpu.make_async_copy(dst_hbm.at[wid, 0], didxb.at[b],
                              isem.at[b]).wait()

    _load(0, 0)
    _load(1, 1)
    _load(2, 2)
    _load(3, 3)
    for bb in range(3):
        _drain_idx(bb)
        _gather(bb)

    def rnd(r, _):
        for b in range(NBUF):
            i = r * NBUF + b
            b3 = (b + 3) % NBUF
            b4 = (b + 4) % NBUF
            _drain_rows(gsem, b)
            _scatter(b)
            def prefetch():
                pl.when(i >= 1)(lambda: _drain_rows(ssem, b4))
                _load(i + 4, b4)
            pl.when(i + 4 < NIT)(prefetch)
            def launch_gather():
                _drain_idx(b3)
                _gather(b3)
            pl.when(i + 3 < NIT)(launch_gather)
        return 0
    lax.fori_loop(0, NIT // NBUF, rnd, 0)

    for b in range(NBUF):
        _drain_rows(ssem, b)

    plsc.subcore_barrier()
    pltpu.sync_copy(m_sh.at[pl.ds(s * RPT, RPT)],
                    out_hbm.at[c, pl.ds(s * RPT, RPT)])


def _aggregate(hs, src3, dst3):
    mesh = plsc.VectorSubcoreMesh(core_axis_name="c", subcore_axis_name="s")
    return pl.kernel(
        _agg_body,
        out_type=jax.ShapeDtypeStruct((NC, MPAD, D), jnp.float32),
        mesh=mesh,
        compiler_params=_SC_PARAMS,
        scratch_types=[
            pltpu.VMEM_SHARED((MPAD, D), jnp.float32),
            pltpu.VMEM((NBUF, CH), jnp.int32),
            pltpu.VMEM((NBUF, CH), jnp.int32),
            pltpu.VMEM((NBUF, CH, D), jnp.float32),
            pltpu.VMEM((64, D), jnp.float32),
            pltpu.SemaphoreType.DMA((NBUF,)),
            pltpu.SemaphoreType.DMA((NBUF,)),
            pltpu.SemaphoreType.DMA((NBUF,)),
        ],
    )(hs, src3, dst3)


def _dense_body(x_ref, wu1_ref, w2_ref, wu2_ref, bu1_ref, bu2_ref,
                hu_ref, uo_ref):
    x = x_ref[...]
    u = jnp.maximum(jnp.dot(x, wu1_ref[...],
                            preferred_element_type=jnp.float32)
                    + bu1_ref[...], 0.0)
    hu_ref[...] = jnp.dot(u, w2_ref[...], preferred_element_type=jnp.float32)
    uo_ref[...] = jnp.maximum(jnp.dot(u, wu2_ref[...],
                                      preferred_element_type=jnp.float32)
                              + bu2_ref[...], 0.0)


def _dense(x, wu1, w2, wu2, bu1, bu2):
    r = 1000
    return pl.pallas_call(
        _dense_body,
        grid=(N // r,),
        in_specs=[
            pl.BlockSpec((r, D), lambda i: (i, 0)),
            pl.BlockSpec((D, D), lambda i: (0, 0)),
            pl.BlockSpec((D, D), lambda i: (0, 0)),
            pl.BlockSpec((D, D), lambda i: (0, 0)),
            pl.BlockSpec((1, D), lambda i: (0, 0)),
            pl.BlockSpec((1, D), lambda i: (0, 0)),
        ],
        out_specs=[
            pl.BlockSpec((r, D), lambda i: (i, 0)),
            pl.BlockSpec((r, D), lambda i: (i, 0)),
        ],
        out_shape=[
            jax.ShapeDtypeStruct((N, D), jnp.float32),
            jax.ShapeDtypeStruct((N, D), jnp.float32),
        ],
    )(x, wu1, w2, wu2, bu1, bu2)


def _scale_body(hu_ref, degt_ref, hs_ref):
    od = jnp.sum(degt_ref[:, 0:NW], axis=1)
    onorm = lax.rsqrt(jnp.maximum(od, 1.0))
    hs_ref[...] = hu_ref[...] * onorm[:, None]


def _scale(hu, degt):
    r = 1000
    return pl.pallas_call(
        _scale_body,
        grid=(N // r,),
        in_specs=[
            pl.BlockSpec((r, D), lambda i: (i, 0)),
            pl.BlockSpec((r, 2 * NW), lambda i: (i, 0)),
        ],
        out_specs=pl.BlockSpec((r, D), lambda i: (i, 0)),
        out_shape=jax.ShapeDtypeStruct((N, D), jnp.float32),
    )(hu, degt)


def _final_body(m_ref, degt_ref, b2_ref, out_ref):
    mm = m_ref[0] + m_ref[1]
    ind = jnp.sum(degt_ref[:, NW:2 * NW], axis=1)
    innorm = lax.rsqrt(jnp.maximum(ind, 1.0))
    out_ref[...] = jnp.maximum(mm * innorm[:, None] + b2_ref[...], 0.0)


def _finalize(m, degt, b2):
    r = 1000
    return pl.pallas_call(
        _final_body,
        grid=(N // r,),
        in_specs=[
            pl.BlockSpec((NC, r, D), lambda i: (0, i, 0)),
            pl.BlockSpec((r, 2 * NW), lambda i: (i, 0)),
            pl.BlockSpec((1, D), lambda i: (0, 0)),
        ],
        out_specs=pl.BlockSpec((r, D), lambda i: (i, 0)),
        out_shape=jax.ShapeDtypeStruct((N, D), jnp.float32),
    )(m, degt, b2)


def kernel(x_user, edge_index0, edge_index1, W1, b1, W2, b2, Wu1, bu1,
           Wu2, bu2):
    src = edge_index1[0]
    dst = edge_index1[1]

    deg = _degrees(src, dst)
    degt = deg.reshape(2 * NW, N).T

    hu, uo = _dense(x_user, Wu1, W2, Wu2, bu1.reshape(1, D),
                    bu2.reshape(1, D))
    hs = _scale(hu, degt)

    m = _aggregate(hs, src.reshape(NW, NIT, CH),
                   dst.reshape(NW, NIT, CH))
    game = _finalize(m, degt, b2.reshape(1, D))

    return (game, uo)
